# Initial kernel scaffold; baseline (speedup 1.0000x reference)
import functools

import jax
import jax.numpy as jnp
from jax import lax
from jax.experimental import pallas as pl
from jax.experimental.pallas import tpu as pltpu
from jax.experimental.pallas import tpu_sc as plsc

N = 10000
E = 320000
F = 128
H = 8
D = 16
SCALE = float(D) ** -0.5
CHUNK = 128
NCHUNKS = E // CHUNK
NTILES = 32
ROWS_PER_TILE = N // 16

_f32 = jnp.float32
_i32 = jnp.int32


def _proj_body(x_ref, w_ref, b_ref, q_ref, k_ref, v_ref):
    big = jnp.dot(x_ref[...], w_ref[...], preferred_element_type=_f32)
    big = big + b_ref[...]
    q_ref[...] = big[:, 0:128]
    k_ref[...] = big[:, 128:256]
    v_ref[...] = big[:, 256:384]


def _projections(x, wt, b):
    blk = 1000
    grid = (N // blk,)
    out = jax.ShapeDtypeStruct((N, F), _f32)
    return pl.pallas_call(
        _proj_body,
        grid=grid,
        in_specs=[
            pl.BlockSpec((blk, F), lambda i: (i, 0)),
            pl.BlockSpec((F, 3 * F), lambda i: (0, 0)),
            pl.BlockSpec((1, 3 * F), lambda i: (0, 0)),
        ],
        out_specs=[
            pl.BlockSpec((blk, F), lambda i: (i, 0)),
            pl.BlockSpec((blk, F), lambda i: (i, 0)),
            pl.BlockSpec((blk, F), lambda i: (i, 0)),
        ],
        out_shape=[out, out, out],
    )(x, wt, b)


def _recip_body(den_ref, o_ref):
    s = jnp.sum(den_ref[...], axis=0)
    o_ref[...] = 0.125 / (s + 1e-16)


def _recip(den_parts):
    blk = (N * H) // 10
    return pl.pallas_call(
        _recip_body,
        grid=(10,),
        in_specs=[pl.BlockSpec((NTILES, blk), lambda i: (0, i))],
        out_specs=pl.BlockSpec((1, blk), lambda i: (i, 0)),
        out_shape=jax.ShapeDtypeStruct((10, blk), _f32),
    )(den_parts)


def _sum2_body(p_ref, o_ref):
    o_ref[...] = p_ref[0] + p_ref[1]


def _sum2(parts):
    blk = 1000
    return pl.pallas_call(
        _sum2_body,
        grid=(N // blk,),
        in_specs=[pl.BlockSpec((2, blk, F), lambda i: (0, i, 0))],
        out_specs=pl.BlockSpec((blk, F), lambda i: (i, 0)),
        out_shape=jax.ShapeDtypeStruct((N, F), _f32),
    )(parts)


_MESH = plsc.VectorSubcoreMesh(core_axis_name="c", subcore_axis_name="s")


@functools.partial(
    pl.kernel,
    out_type=(
        jax.ShapeDtypeStruct((NCHUNKS, H, CHUNK), _f32),
        jax.ShapeDtypeStruct((NTILES, N * H), _f32),
    ),
    mesh=_MESH,
    scratch_types=[
        pltpu.VMEM((CHUNK,), _i32),
        pltpu.VMEM((CHUNK,), _i32),
        pltpu.VMEM((CHUNK, F), _f32),
        pltpu.VMEM((CHUNK, F), _f32),
        pltpu.VMEM((H, CHUNK), _f32),
        pltpu.VMEM((N * H,), _f32),
        pltpu.SemaphoreType.DMA,
        pltpu.SemaphoreType.DMA,
    ],
)
def _edge_scores(q_hbm, k_hbm, adj_hbm, ex_hbm, den_hbm,
                 srcv, dstv, qbuf, kbuf, exbuf, denbuf, sem1, sem2):
    wid = lax.axis_index("s") * 2 + lax.axis_index("c")
    iota16 = lax.iota(_i32, 16)

    @pl.loop(0, (N * H) // 16)
    def _zero(i):
        denbuf[pl.ds(i * 16, 16)] = jnp.zeros((16,), _f32)

    cnt = jnp.where(wid < NCHUNKS % NTILES, NCHUNKS // NTILES + 1,
                    NCHUNKS // NTILES)

    @pl.loop(0, cnt)
    def _chunk(i):
        c = wid + i * NTILES
        off = c * CHUNK
        pltpu.sync_copy(adj_hbm.at[0, pl.ds(off, CHUNK)], srcv)
        pltpu.sync_copy(adj_hbm.at[1, pl.ds(off, CHUNK)], dstv)
        cp_q = pltpu.async_copy(q_hbm.at[srcv], qbuf, sem1)
        cp_k = pltpu.async_copy(k_hbm.at[dstv], kbuf, sem2)
        cp_q.wait()
        cp_k.wait()

        @pl.loop(0, CHUNK // 16)
        def _group(eg):
            rows = eg * 16 + iota16
            dst16 = dstv[pl.ds(eg * 16, 16)]

            @pl.loop(0, H)
            def _head(h):
                colbase = h * 16
                acc = None
                for f in range(D):
                    col = jnp.full((16,), colbase + f, _i32)
                    qv = plsc.load_gather(qbuf, [rows, col])
                    kv = plsc.load_gather(kbuf, [rows, col])
                    t = qv * kv
                    acc = t if acc is None else acc + t
                exv = jnp.exp(acc)
                exbuf[h, pl.ds(eg * 16, 16)] = exv
                plsc.addupdate_scatter(denbuf, [dst16 * H + h], exv)

        pltpu.sync_copy(exbuf, ex_hbm.at[c])

    pltpu.sync_copy(denbuf, den_hbm.at[wid])


@functools.partial(
    pl.kernel,
    out_type=(
        jax.ShapeDtypeStruct((E,), _f32),
        jax.ShapeDtypeStruct((2, N, F), _f32),
    ),
    mesh=_MESH,
    scratch_types=[
        pltpu.VMEM((CHUNK,), _i32),
        pltpu.VMEM((CHUNK,), _i32),
        pltpu.VMEM((H, CHUNK), _f32),
        pltpu.VMEM((CHUNK, 16), _f32),
        pltpu.VMEM((CHUNK, F), _f32),
        pltpu.VMEM((CHUNK, F), _f32),
        pltpu.VMEM((CHUNK,), _f32),
        pltpu.VMEM((125, F), _f32),
        pltpu.VMEM_SHARED((N, F), _f32),
        pltpu.SemaphoreType.DMA,
        pltpu.SemaphoreType.DMA,
    ],
)
def _aggregate(ex_hbm, recip_hbm, vf_hbm, adj_hbm, att_hbm, outp_hbm,
               srcv, dstv, exbuf, recipg, vfbuf, msgbuf, attbuf, zbuf,
               acc_shared, sem1, sem2):
    cid = lax.axis_index("c")
    sid = lax.axis_index("s")
    wid = sid * 2 + cid
    iota16 = lax.iota(_i32, 16)

    @pl.loop(0, 125)
    def _zr(r):
        @pl.loop(0, F // 16)
        def _zc(j):
            zbuf[r, pl.ds(j * 16, 16)] = jnp.zeros((16,), _f32)

    for t in range(5):
        pltpu.sync_copy(
            zbuf, acc_shared.at[pl.ds(sid * ROWS_PER_TILE + t * 125, 125), :])
    plsc.subcore_barrier()

    cnt = jnp.where(wid < NCHUNKS % NTILES, NCHUNKS // NTILES + 1,
                    NCHUNKS // NTILES)

    @pl.loop(0, cnt)
    def _chunk(i):
        c = wid + i * NTILES
        off = c * CHUNK
        pltpu.sync_copy(adj_hbm.at[0, pl.ds(off, CHUNK)], srcv)
        pltpu.sync_copy(adj_hbm.at[1, pl.ds(off, CHUNK)], dstv)
        cp_r = pltpu.async_copy(recip_hbm.at[dstv], recipg, sem1)
        cp_v = pltpu.async_copy(vf_hbm.at[dstv], vfbuf, sem2)
        pltpu.sync_copy(ex_hbm.at[c], exbuf)
        cp_r.wait()
        cp_v.wait()

        @pl.loop(0, CHUNK // 16)
        def _group(eg):
            rows = eg * 16 + iota16
            acc = jnp.zeros((16,), _f32)
            for h in range(H):
                exv = exbuf[h, pl.ds(eg * 16, 16)]
                col = jnp.full((16,), h, _i32)
                rv = plsc.load_gather(recipg, [rows, col])
                acc = acc + exv * rv
            attbuf[pl.ds(eg * 16, 16)] = acc

            @pl.loop(0, 16)
            def _edge(e):
                row = eg * 16 + e
                av = plsc.load_gather(attbuf, [jnp.full((16,), row, _i32)])
                for j in range(F // 16):
                    sl = pl.ds(j * 16, 16)
                    msgbuf[row, sl] = av * vfbuf[row, sl]

        pltpu.sync_copy(attbuf, att_hbm.at[pl.ds(off, CHUNK)])
        pltpu.sync_copy(msgbuf, acc_shared.at[srcv], add=True)

    plsc.subcore_barrier()
    pltpu.sync_copy(
        acc_shared.at[pl.ds(sid * ROWS_PER_TILE, ROWS_PER_TILE), :],
        outp_hbm.at[cid, pl.ds(sid * ROWS_PER_TILE, ROWS_PER_TILE), :])


def kernel(x, adj, Wq, bq, Wk, bk, Wv, bv):
    perm = jnp.asarray([(j % H) * D + j // H for j in range(F)], dtype=_i32)
    w = jnp.concatenate([Wq * SCALE, Wk, Wv[perm, :]], axis=0)
    b = jnp.concatenate([bq * SCALE, bk, bv[perm]], axis=0).reshape(1, 3 * F)

    q, k, vf = _projections(x, w.T, b)
    ex, den_parts = _edge_scores(q, k, adj)
    recip = _recip(den_parts).reshape(N, H)
    recip16 = jnp.concatenate([recip, jnp.zeros((N, H), _f32)], axis=1)
    att, out_parts = _aggregate(ex, recip16, vf, adj)
    out = _sum2(out_parts)
    att = att.reshape(E)
    return (out, att, att)


# speedup vs baseline: 3.5906x; 3.5906x over previous
import functools

import jax
import jax.numpy as jnp
from jax import lax
from jax.experimental import pallas as pl
from jax.experimental.pallas import tpu as pltpu
from jax.experimental.pallas import tpu_sc as plsc

N = 10000
E = 320000
F = 128
H = 8
D = 16
SCALE = float(D) ** -0.5
CHUNK = 128
NCHUNKS = E // CHUNK
NTILES = 32
ROWS_PER_TILE = N // 16

_f32 = jnp.float32
_i32 = jnp.int32


def _proj_body(x_ref, w_ref, b_ref, q_ref, k_ref, v_ref):
    big = jnp.dot(x_ref[...], w_ref[...], preferred_element_type=_f32)
    big = big + b_ref[...]
    q_ref[...] = big[:, 0:128]
    k_ref[...] = big[:, 128:256]
    v_ref[...] = big[:, 256:384]


def _projections(x, wt, b):
    blk = 1000
    grid = (N // blk,)
    out = jax.ShapeDtypeStruct((N, F), _f32)
    return pl.pallas_call(
        _proj_body,
        grid=grid,
        in_specs=[
            pl.BlockSpec((blk, F), lambda i: (i, 0)),
            pl.BlockSpec((F, 3 * F), lambda i: (0, 0)),
            pl.BlockSpec((1, 3 * F), lambda i: (0, 0)),
        ],
        out_specs=[
            pl.BlockSpec((blk, F), lambda i: (i, 0)),
            pl.BlockSpec((blk, F), lambda i: (i, 0)),
            pl.BlockSpec((blk, F), lambda i: (i, 0)),
        ],
        out_shape=[out, out, out],
    )(x, wt, b)


def _recip_body(den_ref, o_ref):
    s = jnp.sum(den_ref[...], axis=0, keepdims=True)
    o_ref[...] = 0.125 / (s + 1e-16)


def _recip(den_parts):
    blk = 16000
    return pl.pallas_call(
        _recip_body,
        grid=((N * H) // blk,),
        in_specs=[pl.BlockSpec((NTILES, blk), lambda i: (0, i))],
        out_specs=pl.BlockSpec((1, blk), lambda i: (0, i)),
        out_shape=jax.ShapeDtypeStruct((1, N * H), _f32),
    )(den_parts)


def _sum2_body(p_ref, o_ref):
    o_ref[...] = p_ref[0] + p_ref[1]


def _sum2(parts):
    blk = 1000
    return pl.pallas_call(
        _sum2_body,
        grid=(N // blk,),
        in_specs=[pl.BlockSpec((2, blk, F), lambda i: (0, i, 0))],
        out_specs=pl.BlockSpec((blk, F), lambda i: (i, 0)),
        out_shape=jax.ShapeDtypeStruct((N, F), _f32),
    )(parts)


_MESH = plsc.VectorSubcoreMesh(core_axis_name="c", subcore_axis_name="s")
_SC_PARAMS = pltpu.CompilerParams(needs_layout_passes=False)


@functools.partial(
    pl.kernel,
    out_type=(
        jax.ShapeDtypeStruct((NCHUNKS, H, CHUNK), _f32),
        jax.ShapeDtypeStruct((NTILES, N * H), _f32),
    ),
    mesh=_MESH,
    compiler_params=_SC_PARAMS,
    scratch_types=[
        pltpu.VMEM((CHUNK,), _i32),
        pltpu.VMEM((CHUNK,), _i32),
        pltpu.VMEM((CHUNK, F), _f32),
        pltpu.VMEM((CHUNK, F), _f32),
        pltpu.VMEM((H, CHUNK), _f32),
        pltpu.VMEM((N * H,), _f32),
        pltpu.SemaphoreType.DMA,
        pltpu.SemaphoreType.DMA,
    ],
)
def _edge_scores(q_hbm, k_hbm, adj_hbm, ex_hbm, den_hbm,
                 srcv, dstv, qbuf, kbuf, exbuf, denbuf, sem1, sem2):
    wid = lax.axis_index("s") * 2 + lax.axis_index("c")
    iota16 = lax.iota(_i32, 16)

    @pl.loop(0, (N * H) // 16)
    def _zero(i):
        denbuf[pl.ds(i * 16, 16)] = jnp.zeros((16,), _f32)

    cnt = jnp.where(wid < NCHUNKS % NTILES, NCHUNKS // NTILES + 1,
                    NCHUNKS // NTILES)

    @pl.loop(0, cnt)
    def _chunk(i):
        c = wid + i * NTILES
        off = c * CHUNK
        pltpu.sync_copy(adj_hbm.at[0, pl.ds(off, CHUNK)], srcv)
        pltpu.sync_copy(adj_hbm.at[1, pl.ds(off, CHUNK)], dstv)
        cp_q = pltpu.async_copy(q_hbm.at[srcv], qbuf, sem1)
        cp_k = pltpu.async_copy(k_hbm.at[dstv], kbuf, sem2)
        cp_q.wait()
        cp_k.wait()

        @pl.loop(0, CHUNK // 16)
        def _group(eg):
            rows = eg * 16 + iota16
            dst16 = dstv[pl.ds(eg * 16, 16)]

            @pl.loop(0, H)
            def _head(h):
                colbase = h * 16
                acc = None
                for f in range(D):
                    col = jnp.full((16,), colbase + f, _i32)
                    qv = plsc.load_gather(qbuf, [rows, col])
                    kv = plsc.load_gather(kbuf, [rows, col])
                    t = qv * kv
                    acc = t if acc is None else acc + t
                exv = jnp.exp(acc)
                exbuf[h, pl.ds(eg * 16, 16)] = exv
                plsc.addupdate_scatter(denbuf, [dst16 * H + h], exv)

        pltpu.sync_copy(exbuf, ex_hbm.at[c])

    pltpu.sync_copy(denbuf, den_hbm.at[wid])


@functools.partial(
    pl.kernel,
    out_type=jax.ShapeDtypeStruct((E,), _f32),
    mesh=_MESH,
    compiler_params=_SC_PARAMS,
    scratch_types=[
        pltpu.VMEM((CHUNK,), _i32),
        pltpu.VMEM((H, CHUNK), _f32),
        pltpu.VMEM((N * H,), _f32),
        pltpu.VMEM((CHUNK,), _f32),
        pltpu.SemaphoreType.DMA,
    ],
)
def _normalize(ex_hbm, recip_hbm, adj_hbm, att_hbm,
               dstv, exbuf, rectab, attbuf, sem1):
    wid = lax.axis_index("s") * 2 + lax.axis_index("c")
    iota16 = lax.iota(_i32, 16)

    pltpu.async_copy(recip_hbm, rectab, sem1).wait()

    cnt = jnp.where(wid < NCHUNKS % NTILES, NCHUNKS // NTILES + 1,
                    NCHUNKS // NTILES)

    @pl.loop(0, cnt)
    def _chunk(i):
        c = wid + i * NTILES
        off = c * CHUNK
        pltpu.sync_copy(adj_hbm.at[1, pl.ds(off, CHUNK)], dstv)
        pltpu.sync_copy(ex_hbm.at[c], exbuf)

        @pl.loop(0, CHUNK // 16)
        def _group(eg):
            dst16 = dstv[pl.ds(eg * 16, 16)]
            acc = jnp.zeros((16,), _f32)
            for h in range(H):
                exv = exbuf[h, pl.ds(eg * 16, 16)]
                rv = plsc.load_gather(rectab, [dst16 * H + h])
                acc = acc + exv * rv
            attbuf[pl.ds(eg * 16, 16)] = acc

        pltpu.sync_copy(attbuf, att_hbm.at[pl.ds(off, CHUNK)])


@functools.partial(
    pl.kernel,
    out_type=jax.ShapeDtypeStruct((2, N, F), _f32),
    mesh=_MESH,
    compiler_params=_SC_PARAMS,
    scratch_types=[
        pltpu.VMEM((CHUNK,), _i32),
        pltpu.VMEM((CHUNK,), _i32),
        pltpu.VMEM((CHUNK,), _f32),
        pltpu.VMEM((CHUNK, F), _f32),
        pltpu.VMEM((CHUNK, F), _f32),
        pltpu.VMEM_SHARED((N, F), _f32),
        pltpu.SemaphoreType.DMA,
    ],
)
def _aggregate(att_hbm, vf_hbm, adj_hbm, outp_hbm,
               srcv, dstv, attv, vfbuf, msgbuf, acc_shared, sem2):
    cid = lax.axis_index("c")
    sid = lax.axis_index("s")
    wid = sid * 2 + cid
    iota16 = lax.iota(_i32, 16)

    base = sid * 624

    @pl.loop(0, CHUNK)
    def _zr(r):
        @pl.loop(0, F // 16)
        def _zc(j):
            msgbuf[r, pl.ds(j * 16, 16)] = jnp.zeros((16,), _f32)

    for t in range(4):
        pltpu.sync_copy(
            msgbuf, acc_shared.at[pl.ds(base + t * CHUNK, CHUNK), :])
    pltpu.sync_copy(msgbuf.at[pl.ds(0, 112), :],
                    acc_shared.at[pl.ds(base + 512, 112), :])

    @pl.when(sid == 15)
    def _ztail():
        pltpu.sync_copy(msgbuf.at[pl.ds(0, 16), :],
                        acc_shared.at[pl.ds(9984, 16), :])

    plsc.subcore_barrier()

    cnt = jnp.where(wid < NCHUNKS % NTILES, NCHUNKS // NTILES + 1,
                    NCHUNKS // NTILES)

    @pl.loop(0, cnt)
    def _chunk(i):
        c = wid + i * NTILES
        off = c * CHUNK
        pltpu.sync_copy(adj_hbm.at[0, pl.ds(off, CHUNK)], srcv)
        pltpu.sync_copy(adj_hbm.at[1, pl.ds(off, CHUNK)], dstv)
        cp_v = pltpu.async_copy(vf_hbm.at[dstv], vfbuf, sem2)
        pltpu.sync_copy(att_hbm.at[pl.ds(off, CHUNK)], attv)
        cp_v.wait()

        @pl.loop(0, CHUNK)
        def _edge(row):
            av = plsc.load_gather(attv, [jnp.full((16,), row, _i32)])
            for j in range(F // 16):
                sl = pl.ds(j * 16, 16)
                msgbuf[row, sl] = av * vfbuf[row, sl]

        pltpu.sync_copy(msgbuf, acc_shared.at[srcv], add=True)

    plsc.subcore_barrier()
    pltpu.sync_copy(
        acc_shared.at[pl.ds(base, 624), :],
        outp_hbm.at[cid, pl.ds(base, 624), :])

    @pl.when(sid == 15)
    def _dtail():
        pltpu.sync_copy(
            acc_shared.at[pl.ds(9984, 16), :],
            outp_hbm.at[cid, pl.ds(9984, 16), :])


def kernel(x, adj, Wq, bq, Wk, bk, Wv, bv):
    perm = jnp.asarray([(j % H) * D + j // H for j in range(F)], dtype=_i32)
    w = jnp.concatenate([Wq * SCALE, Wk, Wv[perm, :]], axis=0)
    b = jnp.concatenate([bq * SCALE, bk, bv[perm]], axis=0).reshape(1, 3 * F)

    q, k, vf = _projections(x, w.T, b)
    ex, den_parts = _edge_scores(q, k, adj)
    recip = _recip(den_parts).reshape(N * H)
    att = _normalize(ex, recip, adj)
    out_parts = _aggregate(att, vf, adj)
    out = _sum2(out_parts)
    return (out, att, att)


# speedup vs baseline: 4.0735x; 1.1345x over previous
import functools

import jax
import jax.numpy as jnp
from jax import lax
from jax.experimental import pallas as pl
from jax.experimental.pallas import tpu as pltpu
from jax.experimental.pallas import tpu_sc as plsc

N = 10000
E = 320000
F = 128
H = 8
D = 16
SCALE = float(D) ** -0.5
CHUNK = 128
NCHUNKS = E // CHUNK
CH_A = 64
NCH_A = E // CH_A
NTILES = 32
ROWS_PER_TILE = N // 16

_f32 = jnp.float32
_i32 = jnp.int32


def _proj_body(x_ref, w_ref, b_ref, q_ref, k_ref, v_ref):
    big = jnp.dot(x_ref[...], w_ref[...], preferred_element_type=_f32)
    big = big + b_ref[...]
    q_ref[...] = big[:, 0:128]
    k_ref[...] = big[:, 128:256]
    v_ref[...] = big[:, 256:384]


def _projections(x, wt, b):
    blk = 1000
    grid = (N // blk,)
    out = jax.ShapeDtypeStruct((N, F), _f32)
    return pl.pallas_call(
        _proj_body,
        grid=grid,
        in_specs=[
            pl.BlockSpec((blk, F), lambda i: (i, 0)),
            pl.BlockSpec((F, 3 * F), lambda i: (0, 0)),
            pl.BlockSpec((1, 3 * F), lambda i: (0, 0)),
        ],
        out_specs=[
            pl.BlockSpec((blk, F), lambda i: (i, 0)),
            pl.BlockSpec((blk, F), lambda i: (i, 0)),
            pl.BlockSpec((blk, F), lambda i: (i, 0)),
        ],
        out_shape=[out, out, out],
    )(x, wt, b)


def _recip_body(den_ref, o_ref):
    s = jnp.sum(den_ref[...], axis=0, keepdims=True)
    o_ref[...] = 0.125 / (s + 1e-16)


def _recip(den_parts):
    blk = 16000
    return pl.pallas_call(
        _recip_body,
        grid=((N * H) // blk,),
        in_specs=[pl.BlockSpec((NTILES, blk), lambda i: (0, i))],
        out_specs=pl.BlockSpec((1, blk), lambda i: (0, i)),
        out_shape=jax.ShapeDtypeStruct((1, N * H), _f32),
    )(den_parts)


def _sum2_body(p_ref, o_ref):
    o_ref[...] = p_ref[0] + p_ref[1]


def _sum2(parts):
    blk = 1000
    return pl.pallas_call(
        _sum2_body,
        grid=(N // blk,),
        in_specs=[pl.BlockSpec((2, blk, F), lambda i: (0, i, 0))],
        out_specs=pl.BlockSpec((blk, F), lambda i: (i, 0)),
        out_shape=jax.ShapeDtypeStruct((N, F), _f32),
    )(parts)


_MESH = plsc.VectorSubcoreMesh(core_axis_name="c", subcore_axis_name="s")
_SC_PARAMS = pltpu.CompilerParams(needs_layout_passes=False)


@functools.partial(
    pl.kernel,
    out_type=(
        jax.ShapeDtypeStruct((NCH_A, H, CH_A), _f32),
        jax.ShapeDtypeStruct((NTILES, N * H), _f32),
    ),
    mesh=_MESH,
    compiler_params=_SC_PARAMS,
    scratch_types=[
        pltpu.VMEM((2, 2, CH_A), _i32),
        pltpu.VMEM((2, CH_A), _i32),
        pltpu.VMEM((2, CH_A, F), _f32),
        pltpu.VMEM((2, CH_A, F), _f32),
        pltpu.VMEM((2, H, CH_A), _f32),
        pltpu.VMEM((N * H,), _f32),
        pltpu.SemaphoreType.DMA,
        pltpu.SemaphoreType.DMA,
        pltpu.SemaphoreType.DMA,
        pltpu.SemaphoreType.DMA,
        pltpu.SemaphoreType.DMA,
        pltpu.SemaphoreType.DMA,
        pltpu.SemaphoreType.DMA,
        pltpu.SemaphoreType.DMA,
    ],
)
def _edge_scores(q_hbm, k_hbm, adj_hbm, ex_hbm, den_hbm,
                 gbuf, dbuf, qbuf, kbuf, exbuf, denbuf,
                 semi0, semi1, semd0, semd1, semg0, semg1, seme0, seme1):
    wid = lax.axis_index("s") * 2 + lax.axis_index("c")
    iota16 = lax.iota(_i32, 16)
    semi = (semi0, semi1)
    semd = (semd0, semd1)
    semg = (semg0, semg1)
    seme = (seme0, seme1)

    @pl.loop(0, (N * H) // 16)
    def _zero(i):
        denbuf[pl.ds(i * 16, 16)] = jnp.zeros((16,), _f32)

    cnt = jnp.where(wid < NCH_A % NTILES, NCH_A // NTILES + 1,
                    NCH_A // NTILES)

    def _chunk_off(i):
        return (wid + i * NTILES) * CH_A

    def _fetch_idx(i, b):
        off = _chunk_off(i)
        pltpu.async_copy(adj_hbm.at[0, pl.ds(off, CH_A)], gbuf.at[b, 0], semi[b])
        pltpu.async_copy(adj_hbm.at[1, pl.ds(off, CH_A)], gbuf.at[b, 1], semi[b])

    def _wait_idx(b):
        pltpu.make_async_copy(
            adj_hbm.at[0, pl.ds(0, CH_A)], gbuf.at[b, 0], semi[b]).wait()
        pltpu.make_async_copy(
            adj_hbm.at[1, pl.ds(0, CH_A)], gbuf.at[b, 1], semi[b]).wait()

    def _start_gather(b):
        pltpu.async_copy(q_hbm.at[gbuf.at[b, 0]], qbuf.at[b], semg[b])
        pltpu.async_copy(k_hbm.at[gbuf.at[b, 1]], kbuf.at[b], semg[b])

    def _wait_gather(b):
        pltpu.make_async_copy(q_hbm.at[gbuf.at[b, 0]], qbuf.at[b], semg[b]).wait()
        pltpu.make_async_copy(k_hbm.at[gbuf.at[b, 1]], kbuf.at[b], semg[b]).wait()

    def _compute(i, b):
        qslot = qbuf.at[b]
        kslot = kbuf.at[b]

        @pl.loop(0, CH_A // 16)
        def _group(eg):
            rows = eg * 16 + iota16
            dst16 = dbuf[b, pl.ds(eg * 16, 16)]

            @pl.loop(0, H)
            def _head(h):
                colbase = h * 16
                acc = None
                for f in range(D):
                    col = jnp.full((16,), colbase + f, _i32)
                    qv = plsc.load_gather(qslot, [rows, col])
                    kv = plsc.load_gather(kslot, [rows, col])
                    t = qv * kv
                    acc = t if acc is None else acc + t
                exv = jnp.exp(acc)
                exbuf[b, h, pl.ds(eg * 16, 16)] = exv
                plsc.addupdate_scatter(denbuf, [dst16 * H + h], exv)

        pltpu.async_copy(exbuf.at[b], ex_hbm.at[wid + i * NTILES], seme[b])

    pltpu.sync_copy(adj_hbm.at[0, pl.ds(_chunk_off(0), CH_A)], gbuf.at[0, 0])
    pltpu.sync_copy(adj_hbm.at[1, pl.ds(_chunk_off(0), CH_A)], gbuf.at[0, 1])
    _start_gather(0)
    _fetch_idx(1, 1)
    pltpu.async_copy(adj_hbm.at[1, pl.ds(_chunk_off(0), CH_A)],
                     dbuf.at[0], semd[0])
    pltpu.async_copy(adj_hbm.at[1, pl.ds(_chunk_off(1), CH_A)],
                     dbuf.at[1], semd[1])

    ceil2 = ((cnt + 1) // 2) * 2

    @pl.loop(0, ceil2, step=2)
    def _ring(g):
        for b in range(2):
            i = g + b
            nb = 1 - b

            @pl.when(i + 1 < cnt)
            def _():
                _wait_idx(nb)
                _start_gather(nb)

            @pl.when(i < cnt)
            def _():
                pltpu.make_async_copy(
                    adj_hbm.at[1, pl.ds(0, CH_A)], dbuf.at[b], semd[b]).wait()
                _wait_gather(b)

            @pl.when(i + 2 < cnt)
            def _():
                _fetch_idx(i + 2, b)

            @pl.when((i >= 2) & (i < cnt))
            def _():
                pltpu.make_async_copy(
                    exbuf.at[b], ex_hbm.at[0], seme[b]).wait()

            @pl.when(i < cnt)
            def _():
                _compute(i, b)

            @pl.when(i + 2 < cnt)
            def _():
                pltpu.async_copy(adj_hbm.at[1, pl.ds(_chunk_off(i + 2), CH_A)],
                                 dbuf.at[b], semd[b])

    pltpu.make_async_copy(exbuf.at[0], ex_hbm.at[0], seme[0]).wait()
    pltpu.make_async_copy(exbuf.at[1], ex_hbm.at[0], seme[1]).wait()

    pltpu.sync_copy(denbuf, den_hbm.at[wid])


@functools.partial(
    pl.kernel,
    out_type=jax.ShapeDtypeStruct((E,), _f32),
    mesh=_MESH,
    compiler_params=_SC_PARAMS,
    scratch_types=[
        pltpu.VMEM((2, CH_A), _i32),
        pltpu.VMEM((2, H, CH_A), _f32),
        pltpu.VMEM((N * H,), _f32),
        pltpu.VMEM((2, CH_A), _f32),
        pltpu.SemaphoreType.DMA,
        pltpu.SemaphoreType.DMA,
        pltpu.SemaphoreType.DMA,
        pltpu.SemaphoreType.DMA,
        pltpu.SemaphoreType.DMA,
    ],
)
def _normalize(ex_hbm, recip_hbm, adj_hbm, att_hbm,
               dstv, exbuf, rectab, attbuf, semr, semx0, semx1, sema0, sema1):
    wid = lax.axis_index("s") * 2 + lax.axis_index("c")
    semx = (semx0, semx1)
    sema = (sema0, sema1)

    cp_rec = pltpu.async_copy(recip_hbm, rectab, semr)

    cnt = jnp.where(wid < NCH_A % NTILES, NCH_A // NTILES + 1,
                    NCH_A // NTILES)

    def _fetch(i, b):
        c = wid + i * NTILES
        pltpu.async_copy(ex_hbm.at[c], exbuf.at[b], semx[b])
        pltpu.async_copy(adj_hbm.at[1, pl.ds(c * CH_A, CH_A)],
                         dstv.at[b], semx[b])

    def _wait_fetch(b):
        pltpu.make_async_copy(ex_hbm.at[0], exbuf.at[b], semx[b]).wait()
        pltpu.make_async_copy(
            adj_hbm.at[1, pl.ds(0, CH_A)], dstv.at[b], semx[b]).wait()

    _fetch(0, 0)
    _fetch(1, 1)
    cp_rec.wait()

    ceil2 = ((cnt + 1) // 2) * 2

    @pl.loop(0, ceil2, step=2)
    def _ring(g):
        for b in range(2):
            i = g + b

            @pl.when(i < cnt)
            def _():
                _wait_fetch(b)

            @pl.when((i >= 2) & (i < cnt))
            def _():
                pltpu.make_async_copy(
                    attbuf.at[b], att_hbm.at[pl.ds(0, CH_A)], sema[b]).wait()

            @pl.when(i < cnt)
            def _():
                @pl.loop(0, CH_A // 16)
                def _group(eg):
                    dst16 = dstv[b, pl.ds(eg * 16, 16)]
                    acc = jnp.zeros((16,), _f32)
                    for h in range(H):
                        exv = exbuf[b, h, pl.ds(eg * 16, 16)]
                        rv = plsc.load_gather(rectab, [dst16 * H + h])
                        acc = acc + exv * rv
                    attbuf[b, pl.ds(eg * 16, 16)] = acc

                c = wid + i * NTILES
                pltpu.async_copy(attbuf.at[b],
                                 att_hbm.at[pl.ds(c * CH_A, CH_A)], sema[b])

            @pl.when(i + 2 < cnt)
            def _():
                _fetch(i + 2, b)

    pltpu.make_async_copy(attbuf.at[0], att_hbm.at[pl.ds(0, CH_A)], sema[0]).wait()
    pltpu.make_async_copy(attbuf.at[1], att_hbm.at[pl.ds(0, CH_A)], sema[1]).wait()


@functools.partial(
    pl.kernel,
    out_type=jax.ShapeDtypeStruct((2, N, F), _f32),
    mesh=_MESH,
    compiler_params=_SC_PARAMS,
    scratch_types=[
        pltpu.VMEM((CHUNK,), _i32),
        pltpu.VMEM((CHUNK,), _i32),
        pltpu.VMEM((CHUNK,), _f32),
        pltpu.VMEM((CHUNK, F), _f32),
        pltpu.VMEM((CHUNK, F), _f32),
        pltpu.VMEM_SHARED((N, F), _f32),
        pltpu.SemaphoreType.DMA,
    ],
)
def _aggregate(att_hbm, vf_hbm, adj_hbm, outp_hbm,
               srcv, dstv, attv, vfbuf, msgbuf, acc_shared, sem2):
    cid = lax.axis_index("c")
    sid = lax.axis_index("s")
    wid = sid * 2 + cid
    iota16 = lax.iota(_i32, 16)

    base = sid * 624

    @pl.loop(0, CHUNK)
    def _zr(r):
        @pl.loop(0, F // 16)
        def _zc(j):
            msgbuf[r, pl.ds(j * 16, 16)] = jnp.zeros((16,), _f32)

    for t in range(4):
        pltpu.sync_copy(
            msgbuf, acc_shared.at[pl.ds(base + t * CHUNK, CHUNK), :])
    pltpu.sync_copy(msgbuf.at[pl.ds(0, 112), :],
                    acc_shared.at[pl.ds(base + 512, 112), :])

    @pl.when(sid == 15)
    def _ztail():
        pltpu.sync_copy(msgbuf.at[pl.ds(0, 16), :],
                        acc_shared.at[pl.ds(9984, 16), :])

    plsc.subcore_barrier()

    cnt = jnp.where(wid < NCHUNKS % NTILES, NCHUNKS // NTILES + 1,
                    NCHUNKS // NTILES)

    @pl.loop(0, cnt)
    def _chunk(i):
        c = wid + i * NTILES
        off = c * CHUNK
        pltpu.sync_copy(adj_hbm.at[0, pl.ds(off, CHUNK)], srcv)
        pltpu.sync_copy(adj_hbm.at[1, pl.ds(off, CHUNK)], dstv)
        cp_v = pltpu.async_copy(vf_hbm.at[dstv], vfbuf, sem2)
        pltpu.sync_copy(att_hbm.at[pl.ds(off, CHUNK)], attv)
        cp_v.wait()

        @pl.loop(0, CHUNK)
        def _edge(row):
            av = plsc.load_gather(attv, [jnp.full((16,), row, _i32)])
            for j in range(F // 16):
                sl = pl.ds(j * 16, 16)
                msgbuf[row, sl] = av * vfbuf[row, sl]

        pltpu.sync_copy(msgbuf, acc_shared.at[srcv], add=True)

    plsc.subcore_barrier()
    pltpu.sync_copy(
        acc_shared.at[pl.ds(base, 624), :],
        outp_hbm.at[cid, pl.ds(base, 624), :])

    @pl.when(sid == 15)
    def _dtail():
        pltpu.sync_copy(
            acc_shared.at[pl.ds(9984, 16), :],
            outp_hbm.at[cid, pl.ds(9984, 16), :])


def kernel(x, adj, Wq, bq, Wk, bk, Wv, bv):
    perm = jnp.asarray([(j % H) * D + j // H for j in range(F)], dtype=_i32)
    w = jnp.concatenate([Wq * SCALE, Wk, Wv[perm, :]], axis=0)
    b = jnp.concatenate([bq * SCALE, bk, bv[perm]], axis=0).reshape(1, 3 * F)

    q, k, vf = _projections(x, w.T, b)
    ex, den_parts = _edge_scores(q, k, adj)
    recip = _recip(den_parts).reshape(N * H)
    att = _normalize(ex, recip, adj)
    out_parts = _aggregate(att, vf, adj)
    out = _sum2(out_parts)
    return (out, att, att)


# speedup vs baseline: 5.2054x; 1.2779x over previous
import functools

import jax
import jax.numpy as jnp
from jax import lax
from jax.experimental import pallas as pl
from jax.experimental.pallas import tpu as pltpu
from jax.experimental.pallas import tpu_sc as plsc

N = 10000
E = 320000
F = 128
H = 8
D = 16
SCALE = float(D) ** -0.5
CHUNK = 128
NCHUNKS = E // CHUNK
CH_A = 64
NCH_A = E // CH_A
NTILES = 32
ROWS_PER_TILE = N // 16

_f32 = jnp.float32
_i32 = jnp.int32


def _proj_body(x_ref, w_ref, b_ref, q_ref, k_ref, v_ref):
    big = jnp.dot(x_ref[...], w_ref[...], preferred_element_type=_f32)
    big = big + b_ref[...]
    q_ref[...] = big[:, 0:128]
    k_ref[...] = big[:, 128:256]
    v_ref[...] = big[:, 256:384]


def _projections(x, wt, b):
    blk = 1000
    grid = (N // blk,)
    out = jax.ShapeDtypeStruct((N, F), _f32)
    return pl.pallas_call(
        _proj_body,
        grid=grid,
        in_specs=[
            pl.BlockSpec((blk, F), lambda i: (i, 0)),
            pl.BlockSpec((F, 3 * F), lambda i: (0, 0)),
            pl.BlockSpec((1, 3 * F), lambda i: (0, 0)),
        ],
        out_specs=[
            pl.BlockSpec((blk, F), lambda i: (i, 0)),
            pl.BlockSpec((blk, F), lambda i: (i, 0)),
            pl.BlockSpec((blk, F), lambda i: (i, 0)),
        ],
        out_shape=[out, out, out],
    )(x, wt, b)


def _recip_body(den_ref, o_ref):
    s = jnp.sum(den_ref[...], axis=0, keepdims=True)
    o_ref[...] = 0.125 / (s + 1e-16)


def _recip(den_parts):
    blk = 16000
    return pl.pallas_call(
        _recip_body,
        grid=((N * H) // blk,),
        in_specs=[pl.BlockSpec((NTILES, blk), lambda i: (0, i))],
        out_specs=pl.BlockSpec((1, blk), lambda i: (0, i)),
        out_shape=jax.ShapeDtypeStruct((1, N * H), _f32),
    )(den_parts)


def _sum2_body(p_ref, o_ref):
    o_ref[...] = p_ref[0] + p_ref[1]


def _sum2(parts):
    blk = 1000
    return pl.pallas_call(
        _sum2_body,
        grid=(N // blk,),
        in_specs=[pl.BlockSpec((2, blk, F), lambda i: (0, i, 0))],
        out_specs=pl.BlockSpec((blk, F), lambda i: (i, 0)),
        out_shape=jax.ShapeDtypeStruct((N, F), _f32),
    )(parts)


_MESH = plsc.VectorSubcoreMesh(core_axis_name="c", subcore_axis_name="s")
_SC_PARAMS = pltpu.CompilerParams(needs_layout_passes=False)


@functools.partial(
    pl.kernel,
    out_type=(
        jax.ShapeDtypeStruct((NCH_A, H, CH_A), _f32),
        jax.ShapeDtypeStruct((NTILES, N * H), _f32),
    ),
    mesh=_MESH,
    compiler_params=_SC_PARAMS,
    scratch_types=[
        pltpu.VMEM((2, 2, CH_A), _i32),
        pltpu.VMEM((2, CH_A), _i32),
        pltpu.VMEM((2, CH_A, F), _f32),
        pltpu.VMEM((2, CH_A, F), _f32),
        pltpu.VMEM((2, H, CH_A), _f32),
        pltpu.VMEM((N * H,), _f32),
        pltpu.SemaphoreType.DMA,
        pltpu.SemaphoreType.DMA,
        pltpu.SemaphoreType.DMA,
        pltpu.SemaphoreType.DMA,
        pltpu.SemaphoreType.DMA,
        pltpu.SemaphoreType.DMA,
        pltpu.SemaphoreType.DMA,
        pltpu.SemaphoreType.DMA,
    ],
)
def _edge_scores(q_hbm, k_hbm, adj_hbm, ex_hbm, den_hbm,
                 gbuf, dbuf, qbuf, kbuf, exbuf, denbuf,
                 semi0, semi1, semd0, semd1, semg0, semg1, seme0, seme1):
    wid = lax.axis_index("s") * 2 + lax.axis_index("c")
    iota16 = lax.iota(_i32, 16)
    semi = (semi0, semi1)
    semd = (semd0, semd1)
    semg = (semg0, semg1)
    seme = (seme0, seme1)

    @pl.loop(0, (N * H) // 16)
    def _zero(i):
        denbuf[pl.ds(i * 16, 16)] = jnp.zeros((16,), _f32)

    cnt = jnp.where(wid < NCH_A % NTILES, NCH_A // NTILES + 1,
                    NCH_A // NTILES)

    def _chunk_off(i):
        return (wid + i * NTILES) * CH_A

    def _fetch_idx(i, b):
        off = _chunk_off(i)
        pltpu.async_copy(adj_hbm.at[0, pl.ds(off, CH_A)], gbuf.at[b, 0], semi[b])
        pltpu.async_copy(adj_hbm.at[1, pl.ds(off, CH_A)], gbuf.at[b, 1], semi[b])

    def _wait_idx(b):
        pltpu.make_async_copy(
            adj_hbm.at[0, pl.ds(0, CH_A)], gbuf.at[b, 0], semi[b]).wait()
        pltpu.make_async_copy(
            adj_hbm.at[1, pl.ds(0, CH_A)], gbuf.at[b, 1], semi[b]).wait()

    def _start_gather(b):
        pltpu.async_copy(q_hbm.at[gbuf.at[b, 0]], qbuf.at[b], semg[b])
        pltpu.async_copy(k_hbm.at[gbuf.at[b, 1]], kbuf.at[b], semg[b])

    def _wait_gather(b):
        pltpu.make_async_copy(q_hbm.at[gbuf.at[b, 0]], qbuf.at[b], semg[b]).wait()
        pltpu.make_async_copy(k_hbm.at[gbuf.at[b, 1]], kbuf.at[b], semg[b]).wait()

    def _compute(i, b):
        qslot = qbuf.at[b]
        kslot = kbuf.at[b]

        @pl.loop(0, CH_A // 16)
        def _group(eg):
            rows = eg * 16 + iota16
            dst16 = dbuf[b, pl.ds(eg * 16, 16)]

            @pl.loop(0, H)
            def _head(h):
                colbase = h * 16
                accs = [None] * 4
                for f in range(0, D, 4):
                    qvs = []
                    kvs = []
                    for u in range(4):
                        col = jnp.full((16,), colbase + f + u, _i32)
                        qvs.append(plsc.load_gather(qslot, [rows, col]))
                        kvs.append(plsc.load_gather(kslot, [rows, col]))
                    for u in range(4):
                        t = qvs[u] * kvs[u]
                        accs[u] = t if accs[u] is None else accs[u] + t
                exv = jnp.exp((accs[0] + accs[1]) + (accs[2] + accs[3]))
                exbuf[b, h, pl.ds(eg * 16, 16)] = exv
                plsc.addupdate_scatter(denbuf, [dst16 * H + h], exv)

        pltpu.async_copy(exbuf.at[b], ex_hbm.at[wid + i * NTILES], seme[b])

    pltpu.sync_copy(adj_hbm.at[0, pl.ds(_chunk_off(0), CH_A)], gbuf.at[0, 0])
    pltpu.sync_copy(adj_hbm.at[1, pl.ds(_chunk_off(0), CH_A)], gbuf.at[0, 1])
    _start_gather(0)
    _fetch_idx(1, 1)
    pltpu.async_copy(adj_hbm.at[1, pl.ds(_chunk_off(0), CH_A)],
                     dbuf.at[0], semd[0])
    pltpu.async_copy(adj_hbm.at[1, pl.ds(_chunk_off(1), CH_A)],
                     dbuf.at[1], semd[1])

    ceil2 = ((cnt + 1) // 2) * 2

    @pl.loop(0, ceil2, step=2)
    def _ring(g):
        for b in range(2):
            i = g + b
            nb = 1 - b

            @pl.when(i + 1 < cnt)
            def _():
                _wait_idx(nb)
                _start_gather(nb)

            @pl.when(i < cnt)
            def _():
                pltpu.make_async_copy(
                    adj_hbm.at[1, pl.ds(0, CH_A)], dbuf.at[b], semd[b]).wait()
                _wait_gather(b)

            @pl.when(i + 2 < cnt)
            def _():
                _fetch_idx(i + 2, b)

            @pl.when((i >= 2) & (i < cnt))
            def _():
                pltpu.make_async_copy(
                    exbuf.at[b], ex_hbm.at[0], seme[b]).wait()

            @pl.when(i < cnt)
            def _():
                _compute(i, b)

            @pl.when(i + 2 < cnt)
            def _():
                pltpu.async_copy(adj_hbm.at[1, pl.ds(_chunk_off(i + 2), CH_A)],
                                 dbuf.at[b], semd[b])

    pltpu.make_async_copy(exbuf.at[0], ex_hbm.at[0], seme[0]).wait()
    pltpu.make_async_copy(exbuf.at[1], ex_hbm.at[0], seme[1]).wait()

    pltpu.sync_copy(denbuf, den_hbm.at[wid])


@functools.partial(
    pl.kernel,
    out_type=jax.ShapeDtypeStruct((E,), _f32),
    mesh=_MESH,
    compiler_params=_SC_PARAMS,
    scratch_types=[
        pltpu.VMEM((2, CH_A), _i32),
        pltpu.VMEM((2, H, CH_A), _f32),
        pltpu.VMEM((N * H,), _f32),
        pltpu.VMEM((2, CH_A), _f32),
        pltpu.SemaphoreType.DMA,
        pltpu.SemaphoreType.DMA,
        pltpu.SemaphoreType.DMA,
        pltpu.SemaphoreType.DMA,
        pltpu.SemaphoreType.DMA,
    ],
)
def _normalize(ex_hbm, recip_hbm, adj_hbm, att_hbm,
               dstv, exbuf, rectab, attbuf, semr, semx0, semx1, sema0, sema1):
    wid = lax.axis_index("s") * 2 + lax.axis_index("c")
    semx = (semx0, semx1)
    sema = (sema0, sema1)

    cp_rec = pltpu.async_copy(recip_hbm, rectab, semr)

    cnt = jnp.where(wid < NCH_A % NTILES, NCH_A // NTILES + 1,
                    NCH_A // NTILES)

    def _fetch(i, b):
        c = wid + i * NTILES
        pltpu.async_copy(ex_hbm.at[c], exbuf.at[b], semx[b])
        pltpu.async_copy(adj_hbm.at[1, pl.ds(c * CH_A, CH_A)],
                         dstv.at[b], semx[b])

    def _wait_fetch(b):
        pltpu.make_async_copy(ex_hbm.at[0], exbuf.at[b], semx[b]).wait()
        pltpu.make_async_copy(
            adj_hbm.at[1, pl.ds(0, CH_A)], dstv.at[b], semx[b]).wait()

    _fetch(0, 0)
    _fetch(1, 1)
    cp_rec.wait()

    ceil2 = ((cnt + 1) // 2) * 2

    @pl.loop(0, ceil2, step=2)
    def _ring(g):
        for b in range(2):
            i = g + b

            @pl.when(i < cnt)
            def _():
                _wait_fetch(b)

            @pl.when((i >= 2) & (i < cnt))
            def _():
                pltpu.make_async_copy(
                    attbuf.at[b], att_hbm.at[pl.ds(0, CH_A)], sema[b]).wait()

            @pl.when(i < cnt)
            def _():
                @pl.loop(0, CH_A // 16)
                def _group(eg):
                    dst16 = dstv[b, pl.ds(eg * 16, 16)]
                    acc = jnp.zeros((16,), _f32)
                    for h in range(H):
                        exv = exbuf[b, h, pl.ds(eg * 16, 16)]
                        rv = plsc.load_gather(rectab, [dst16 * H + h])
                        acc = acc + exv * rv
                    attbuf[b, pl.ds(eg * 16, 16)] = acc

                c = wid + i * NTILES
                pltpu.async_copy(attbuf.at[b],
                                 att_hbm.at[pl.ds(c * CH_A, CH_A)], sema[b])

            @pl.when(i + 2 < cnt)
            def _():
                _fetch(i + 2, b)

    pltpu.make_async_copy(attbuf.at[0], att_hbm.at[pl.ds(0, CH_A)], sema[0]).wait()
    pltpu.make_async_copy(attbuf.at[1], att_hbm.at[pl.ds(0, CH_A)], sema[1]).wait()


@functools.partial(
    pl.kernel,
    out_type=jax.ShapeDtypeStruct((2, N, F), _f32),
    mesh=_MESH,
    compiler_params=_SC_PARAMS,
    scratch_types=[
        pltpu.VMEM((4, CH_A), _i32),
        pltpu.VMEM((2, CH_A), _i32),
        pltpu.VMEM((2, CH_A), _f32),
        pltpu.VMEM((2, CH_A, F), _f32),
        pltpu.VMEM((2, CH_A, F), _f32),
        pltpu.VMEM_SHARED((N, F), _f32),
        pltpu.SemaphoreType.DMA,
        pltpu.SemaphoreType.DMA,
        pltpu.SemaphoreType.DMA,
        pltpu.SemaphoreType.DMA,
        pltpu.SemaphoreType.DMA,
        pltpu.SemaphoreType.DMA,
        pltpu.SemaphoreType.DMA,
        pltpu.SemaphoreType.DMA,
    ],
)
def _aggregate(att_hbm, vf_hbm, adj_hbm, outp_hbm,
               srcv, dstv, attv, vfbuf, msgbuf, acc_shared,
               semd0, semd1, semsr0, semsr1, semv0, semv1, sems0, sems1):
    cid = lax.axis_index("c")
    sid = lax.axis_index("s")
    wid = sid * 2 + cid
    semd = (semd0, semd1)
    semsr = (semsr0, semsr1)
    semv = (semv0, semv1)
    sems = (sems0, sems1)

    base = sid * 624

    @pl.loop(0, CH_A)
    def _zr(r):
        @pl.loop(0, F // 16)
        def _zc(j):
            msgbuf[0, r, pl.ds(j * 16, 16)] = jnp.zeros((16,), _f32)

    for t in range(9):
        pltpu.sync_copy(
            msgbuf.at[0], acc_shared.at[pl.ds(base + t * CH_A, CH_A), :])
    pltpu.sync_copy(msgbuf.at[0, pl.ds(0, 48), :],
                    acc_shared.at[pl.ds(base + 9 * CH_A, 48), :])

    @pl.when(sid == 15)
    def _ztail():
        pltpu.sync_copy(msgbuf.at[0, pl.ds(0, 16), :],
                        acc_shared.at[pl.ds(9984, 16), :])

    plsc.subcore_barrier()

    cnt = jnp.where(wid < NCH_A % NTILES, NCH_A // NTILES + 1,
                    NCH_A // NTILES)

    def _off(i):
        return (wid + i * NTILES) * CH_A

    def _start_gather(i, b):
        pltpu.async_copy(vf_hbm.at[dstv.at[b]], vfbuf.at[b], semv[b])
        pltpu.async_copy(att_hbm.at[pl.ds(_off(i), CH_A)], attv.at[b], semv[b])

    def _wait_gather(b):
        pltpu.make_async_copy(vf_hbm.at[dstv.at[b]], vfbuf.at[b], semv[b]).wait()
        pltpu.make_async_copy(
            att_hbm.at[pl.ds(0, CH_A)], attv.at[b], semv[b]).wait()

    def _wait_scatter(b):
        pltpu.make_async_copy(
            msgbuf.at[b], acc_shared.at[srcv.at[0]], sems[b]).wait()

    pltpu.sync_copy(adj_hbm.at[1, pl.ds(_off(0), CH_A)], dstv.at[0])
    _start_gather(0, 0)
    pltpu.async_copy(adj_hbm.at[1, pl.ds(_off(1), CH_A)], dstv.at[1], semd[1])
    pltpu.async_copy(adj_hbm.at[0, pl.ds(_off(0), CH_A)], srcv.at[0], semsr[0])
    pltpu.async_copy(adj_hbm.at[0, pl.ds(_off(1), CH_A)], srcv.at[1], semsr[1])

    ceil4 = ((cnt + 3) // 4) * 4

    @pl.loop(0, ceil4, step=4)
    def _ring(g):
        for b in range(4):
            i = g + b
            b2 = b % 2
            nb2 = (b + 1) % 2

            @pl.when(i + 1 < cnt)
            def _():
                pltpu.make_async_copy(
                    adj_hbm.at[1, pl.ds(0, CH_A)],
                    dstv.at[nb2], semd[nb2]).wait()
                _start_gather(i + 1, nb2)

            @pl.when(i < cnt)
            def _():
                _wait_gather(b2)

            @pl.when((i >= 2) & (i < cnt))
            def _():
                _wait_scatter(b2)

            @pl.when(i < cnt)
            def _():
                pltpu.make_async_copy(
                    adj_hbm.at[0, pl.ds(0, CH_A)],
                    srcv.at[b], semsr[b2]).wait()

            @pl.when(i + 2 < cnt)
            def _():
                pltpu.async_copy(adj_hbm.at[1, pl.ds(_off(i + 2), CH_A)],
                                 dstv.at[b2], semd[b2])
                pltpu.async_copy(adj_hbm.at[0, pl.ds(_off(i + 2), CH_A)],
                                 srcv.at[(b + 2) % 4], semsr[b2])

            @pl.when(i < cnt)
            def _():
                @pl.loop(0, CH_A)
                def _edge(row):
                    av = plsc.load_gather(
                        attv.at[b2], [jnp.full((16,), row, _i32)])
                    for j in range(F // 16):
                        sl = pl.ds(j * 16, 16)
                        msgbuf[b2, row, sl] = av * vfbuf[b2, row, sl]

                pltpu.async_copy(msgbuf.at[b2], acc_shared.at[srcv.at[b]],
                                 sems[b2], add=True)

    _wait_scatter(0)
    _wait_scatter(1)

    plsc.subcore_barrier()
    pltpu.sync_copy(
        acc_shared.at[pl.ds(base, 624), :],
        outp_hbm.at[cid, pl.ds(base, 624), :])

    @pl.when(sid == 15)
    def _dtail():
        pltpu.sync_copy(
            acc_shared.at[pl.ds(9984, 16), :],
            outp_hbm.at[cid, pl.ds(9984, 16), :])


def kernel(x, adj, Wq, bq, Wk, bk, Wv, bv):
    perm = jnp.asarray([(j % H) * D + j // H for j in range(F)], dtype=_i32)
    w = jnp.concatenate([Wq * SCALE, Wk, Wv[perm, :]], axis=0)
    b = jnp.concatenate([bq * SCALE, bk, bv[perm]], axis=0).reshape(1, 3 * F)

    q, k, vf = _projections(x, w.T, b)
    ex, den_parts = _edge_scores(q, k, adj)
    recip = _recip(den_parts).reshape(N * H)
    att = _normalize(ex, recip, adj)
    out_parts = _aggregate(att, vf, adj)
    out = _sum2(out_parts)
    return (out, att, att)


# speedup vs baseline: 5.2572x; 1.0100x over previous
import functools

import jax
import jax.numpy as jnp
from jax import lax
from jax.experimental import pallas as pl
from jax.experimental.pallas import tpu as pltpu
from jax.experimental.pallas import tpu_sc as plsc

N = 10000
E = 320000
F = 128
H = 8
D = 16
SCALE = float(D) ** -0.5
CHUNK = 128
NCHUNKS = E // CHUNK
CH_A = 64
NCH_A = E // CH_A
NTILES = 32
ROWS_PER_TILE = N // 16

_f32 = jnp.float32
_i32 = jnp.int32


def _proj_body(x_ref, w_ref, b_ref, q_ref, k_ref, v_ref):
    big = jnp.dot(x_ref[...], w_ref[...], preferred_element_type=_f32)
    big = big + b_ref[...]
    q_ref[...] = big[:, 0:128]
    k_ref[...] = big[:, 128:256]
    v_ref[...] = big[:, 256:384]


def _projections(x, wt, b):
    blk = 1000
    grid = (N // blk,)
    out = jax.ShapeDtypeStruct((N, F), _f32)
    return pl.pallas_call(
        _proj_body,
        grid=grid,
        in_specs=[
            pl.BlockSpec((blk, F), lambda i: (i, 0)),
            pl.BlockSpec((F, 3 * F), lambda i: (0, 0)),
            pl.BlockSpec((1, 3 * F), lambda i: (0, 0)),
        ],
        out_specs=[
            pl.BlockSpec((blk, F), lambda i: (i, 0)),
            pl.BlockSpec((blk, F), lambda i: (i, 0)),
            pl.BlockSpec((blk, F), lambda i: (i, 0)),
        ],
        out_shape=[out, out, out],
    )(x, wt, b)


def _recip_body(den_ref, o_ref):
    s = jnp.sum(den_ref[...], axis=0, keepdims=True)
    o_ref[...] = 0.125 / (s + 1e-16)


def _recip(den_parts):
    blk = 16000
    return pl.pallas_call(
        _recip_body,
        grid=((N * H) // blk,),
        in_specs=[pl.BlockSpec((NTILES, blk), lambda i: (0, i))],
        out_specs=pl.BlockSpec((1, blk), lambda i: (0, i)),
        out_shape=jax.ShapeDtypeStruct((1, N * H), _f32),
    )(den_parts)


def _sum2_body(p_ref, o_ref):
    o_ref[...] = p_ref[0] + p_ref[1]


def _sum2(parts):
    blk = 1000
    return pl.pallas_call(
        _sum2_body,
        grid=(N // blk,),
        in_specs=[pl.BlockSpec((2, blk, F), lambda i: (0, i, 0))],
        out_specs=pl.BlockSpec((blk, F), lambda i: (i, 0)),
        out_shape=jax.ShapeDtypeStruct((N, F), _f32),
    )(parts)


_MESH = plsc.VectorSubcoreMesh(core_axis_name="c", subcore_axis_name="s")
_SC_PARAMS = pltpu.CompilerParams(needs_layout_passes=False)


@functools.partial(
    pl.kernel,
    out_type=(
        jax.ShapeDtypeStruct((NCH_A, H, CH_A), _f32),
        jax.ShapeDtypeStruct((NTILES, N * H), _f32),
    ),
    mesh=_MESH,
    compiler_params=_SC_PARAMS,
    scratch_types=[
        pltpu.VMEM((2, 2, CH_A), _i32),
        pltpu.VMEM((2, CH_A), _i32),
        pltpu.VMEM((2, CH_A, F), _f32),
        pltpu.VMEM((2, CH_A, F), _f32),
        pltpu.VMEM((2, H, CH_A), _f32),
        pltpu.VMEM((N * H,), _f32),
        pltpu.SemaphoreType.DMA,
        pltpu.SemaphoreType.DMA,
        pltpu.SemaphoreType.DMA,
        pltpu.SemaphoreType.DMA,
        pltpu.SemaphoreType.DMA,
        pltpu.SemaphoreType.DMA,
        pltpu.SemaphoreType.DMA,
        pltpu.SemaphoreType.DMA,
    ],
)
def _edge_scores(q_hbm, k_hbm, adj_hbm, ex_hbm, den_hbm,
                 gbuf, dbuf, qbuf, kbuf, exbuf, denbuf,
                 semi0, semi1, semd0, semd1, semg0, semg1, seme0, seme1):
    wid = lax.axis_index("s") * 2 + lax.axis_index("c")
    iota16 = lax.iota(_i32, 16)
    semi = (semi0, semi1)
    semd = (semd0, semd1)
    semg = (semg0, semg1)
    seme = (seme0, seme1)

    @pl.loop(0, (N * H) // 16)
    def _zero(i):
        denbuf[pl.ds(i * 16, 16)] = jnp.zeros((16,), _f32)

    cnt = jnp.where(wid < NCH_A % NTILES, NCH_A // NTILES + 1,
                    NCH_A // NTILES)

    def _chunk_off(i):
        return (wid + i * NTILES) * CH_A

    def _fetch_idx(i, b):
        off = _chunk_off(i)
        pltpu.async_copy(adj_hbm.at[0, pl.ds(off, CH_A)], gbuf.at[b, 0], semi[b])
        pltpu.async_copy(adj_hbm.at[1, pl.ds(off, CH_A)], gbuf.at[b, 1], semi[b])

    def _wait_idx(b):
        pltpu.make_async_copy(
            adj_hbm.at[0, pl.ds(0, CH_A)], gbuf.at[b, 0], semi[b]).wait()
        pltpu.make_async_copy(
            adj_hbm.at[1, pl.ds(0, CH_A)], gbuf.at[b, 1], semi[b]).wait()

    def _start_gather(b):
        pltpu.async_copy(q_hbm.at[gbuf.at[b, 0]], qbuf.at[b], semg[b])
        pltpu.async_copy(k_hbm.at[gbuf.at[b, 1]], kbuf.at[b], semg[b])

    def _wait_gather(b):
        pltpu.make_async_copy(q_hbm.at[gbuf.at[b, 0]], qbuf.at[b], semg[b]).wait()
        pltpu.make_async_copy(k_hbm.at[gbuf.at[b, 1]], kbuf.at[b], semg[b]).wait()

    def _compute(i, b):
        qslot = qbuf.at[b]
        kslot = kbuf.at[b]

        @pl.loop(0, CH_A // 16)
        def _group(eg):
            rows = eg * 16 + iota16
            dst16 = dbuf[b, pl.ds(eg * 16, 16)]

            for h in range(H):
                colbase = h * 16
                accs = [None] * 4
                for f in range(0, D, 4):
                    qvs = []
                    kvs = []
                    for u in range(4):
                        col = jnp.full((16,), colbase + f + u, _i32)
                        qvs.append(plsc.load_gather(qslot, [rows, col]))
                        kvs.append(plsc.load_gather(kslot, [rows, col]))
                    for u in range(4):
                        t = qvs[u] * kvs[u]
                        accs[u] = t if accs[u] is None else accs[u] + t
                exv = jnp.exp((accs[0] + accs[1]) + (accs[2] + accs[3]))
                exbuf[b, h, pl.ds(eg * 16, 16)] = exv
                plsc.addupdate_scatter(denbuf, [dst16 * H + h], exv)

        pltpu.async_copy(exbuf.at[b], ex_hbm.at[wid + i * NTILES], seme[b])

    pltpu.sync_copy(adj_hbm.at[0, pl.ds(_chunk_off(0), CH_A)], gbuf.at[0, 0])
    pltpu.sync_copy(adj_hbm.at[1, pl.ds(_chunk_off(0), CH_A)], gbuf.at[0, 1])
    _start_gather(0)
    _fetch_idx(1, 1)
    pltpu.async_copy(adj_hbm.at[1, pl.ds(_chunk_off(0), CH_A)],
                     dbuf.at[0], semd[0])
    pltpu.async_copy(adj_hbm.at[1, pl.ds(_chunk_off(1), CH_A)],
                     dbuf.at[1], semd[1])

    ceil2 = ((cnt + 1) // 2) * 2

    @pl.loop(0, ceil2, step=2)
    def _ring(g):
        for b in range(2):
            i = g + b
            nb = 1 - b

            @pl.when(i + 1 < cnt)
            def _():
                _wait_idx(nb)
                _start_gather(nb)

            @pl.when(i < cnt)
            def _():
                pltpu.make_async_copy(
                    adj_hbm.at[1, pl.ds(0, CH_A)], dbuf.at[b], semd[b]).wait()
                _wait_gather(b)

            @pl.when(i + 2 < cnt)
            def _():
                _fetch_idx(i + 2, b)

            @pl.when((i >= 2) & (i < cnt))
            def _():
                pltpu.make_async_copy(
                    exbuf.at[b], ex_hbm.at[0], seme[b]).wait()

            @pl.when(i < cnt)
            def _():
                _compute(i, b)

            @pl.when(i + 2 < cnt)
            def _():
                pltpu.async_copy(adj_hbm.at[1, pl.ds(_chunk_off(i + 2), CH_A)],
                                 dbuf.at[b], semd[b])

    pltpu.make_async_copy(exbuf.at[0], ex_hbm.at[0], seme[0]).wait()
    pltpu.make_async_copy(exbuf.at[1], ex_hbm.at[0], seme[1]).wait()

    pltpu.sync_copy(denbuf, den_hbm.at[wid])


@functools.partial(
    pl.kernel,
    out_type=jax.ShapeDtypeStruct((E,), _f32),
    mesh=_MESH,
    compiler_params=_SC_PARAMS,
    scratch_types=[
        pltpu.VMEM((2, CH_A), _i32),
        pltpu.VMEM((2, H, CH_A), _f32),
        pltpu.VMEM((N * H,), _f32),
        pltpu.VMEM((2, CH_A), _f32),
        pltpu.SemaphoreType.DMA,
        pltpu.SemaphoreType.DMA,
        pltpu.SemaphoreType.DMA,
        pltpu.SemaphoreType.DMA,
        pltpu.SemaphoreType.DMA,
    ],
)
def _normalize(ex_hbm, recip_hbm, adj_hbm, att_hbm,
               dstv, exbuf, rectab, attbuf, semr, semx0, semx1, sema0, sema1):
    wid = lax.axis_index("s") * 2 + lax.axis_index("c")
    semx = (semx0, semx1)
    sema = (sema0, sema1)

    cp_rec = pltpu.async_copy(recip_hbm, rectab, semr)

    cnt = jnp.where(wid < NCH_A % NTILES, NCH_A // NTILES + 1,
                    NCH_A // NTILES)

    def _fetch(i, b):
        c = wid + i * NTILES
        pltpu.async_copy(ex_hbm.at[c], exbuf.at[b], semx[b])
        pltpu.async_copy(adj_hbm.at[1, pl.ds(c * CH_A, CH_A)],
                         dstv.at[b], semx[b])

    def _wait_fetch(b):
        pltpu.make_async_copy(ex_hbm.at[0], exbuf.at[b], semx[b]).wait()
        pltpu.make_async_copy(
            adj_hbm.at[1, pl.ds(0, CH_A)], dstv.at[b], semx[b]).wait()

    _fetch(0, 0)
    _fetch(1, 1)
    cp_rec.wait()

    ceil2 = ((cnt + 1) // 2) * 2

    @pl.loop(0, ceil2, step=2)
    def _ring(g):
        for b in range(2):
            i = g + b

            @pl.when(i < cnt)
            def _():
                _wait_fetch(b)

            @pl.when((i >= 2) & (i < cnt))
            def _():
                pltpu.make_async_copy(
                    attbuf.at[b], att_hbm.at[pl.ds(0, CH_A)], sema[b]).wait()

            @pl.when(i < cnt)
            def _():
                @pl.loop(0, CH_A // 16)
                def _group(eg):
                    dst16 = dstv[b, pl.ds(eg * 16, 16)]
                    acc = jnp.zeros((16,), _f32)
                    for h in range(H):
                        exv = exbuf[b, h, pl.ds(eg * 16, 16)]
                        rv = plsc.load_gather(rectab, [dst16 * H + h])
                        acc = acc + exv * rv
                    attbuf[b, pl.ds(eg * 16, 16)] = acc

                c = wid + i * NTILES
                pltpu.async_copy(attbuf.at[b],
                                 att_hbm.at[pl.ds(c * CH_A, CH_A)], sema[b])

            @pl.when(i + 2 < cnt)
            def _():
                _fetch(i + 2, b)

    pltpu.make_async_copy(attbuf.at[0], att_hbm.at[pl.ds(0, CH_A)], sema[0]).wait()
    pltpu.make_async_copy(attbuf.at[1], att_hbm.at[pl.ds(0, CH_A)], sema[1]).wait()


@functools.partial(
    pl.kernel,
    out_type=jax.ShapeDtypeStruct((2, N, F), _f32),
    mesh=_MESH,
    compiler_params=_SC_PARAMS,
    scratch_types=[
        pltpu.VMEM((4, CH_A), _i32),
        pltpu.VMEM((2, CH_A), _i32),
        pltpu.VMEM((2, CH_A), _f32),
        pltpu.VMEM((2, CH_A, F), _f32),
        pltpu.VMEM((2, CH_A, F), _f32),
        pltpu.VMEM_SHARED((N, F), _f32),
        pltpu.SemaphoreType.DMA,
        pltpu.SemaphoreType.DMA,
        pltpu.SemaphoreType.DMA,
        pltpu.SemaphoreType.DMA,
        pltpu.SemaphoreType.DMA,
        pltpu.SemaphoreType.DMA,
        pltpu.SemaphoreType.DMA,
        pltpu.SemaphoreType.DMA,
    ],
)
def _aggregate(att_hbm, vf_hbm, adj_hbm, outp_hbm,
               srcv, dstv, attv, vfbuf, msgbuf, acc_shared,
               semd0, semd1, semsr0, semsr1, semv0, semv1, sems0, sems1):
    cid = lax.axis_index("c")
    sid = lax.axis_index("s")
    wid = sid * 2 + cid
    semd = (semd0, semd1)
    semsr = (semsr0, semsr1)
    semv = (semv0, semv1)
    sems = (sems0, sems1)

    base = sid * 624

    @pl.loop(0, CH_A)
    def _zr(r):
        @pl.loop(0, F // 16)
        def _zc(j):
            msgbuf[0, r, pl.ds(j * 16, 16)] = jnp.zeros((16,), _f32)

    for t in range(9):
        pltpu.sync_copy(
            msgbuf.at[0], acc_shared.at[pl.ds(base + t * CH_A, CH_A), :])
    pltpu.sync_copy(msgbuf.at[0, pl.ds(0, 48), :],
                    acc_shared.at[pl.ds(base + 9 * CH_A, 48), :])

    @pl.when(sid == 15)
    def _ztail():
        pltpu.sync_copy(msgbuf.at[0, pl.ds(0, 16), :],
                        acc_shared.at[pl.ds(9984, 16), :])

    plsc.subcore_barrier()

    cnt = jnp.where(wid < NCH_A % NTILES, NCH_A // NTILES + 1,
                    NCH_A // NTILES)

    def _off(i):
        return (wid + i * NTILES) * CH_A

    def _start_gather(i, b):
        pltpu.async_copy(vf_hbm.at[dstv.at[b]], vfbuf.at[b], semv[b])
        pltpu.async_copy(att_hbm.at[pl.ds(_off(i), CH_A)], attv.at[b], semv[b])

    def _wait_gather(b):
        pltpu.make_async_copy(vf_hbm.at[dstv.at[b]], vfbuf.at[b], semv[b]).wait()
        pltpu.make_async_copy(
            att_hbm.at[pl.ds(0, CH_A)], attv.at[b], semv[b]).wait()

    def _wait_scatter(b):
        pltpu.make_async_copy(
            msgbuf.at[b], acc_shared.at[srcv.at[0]], sems[b]).wait()

    pltpu.sync_copy(adj_hbm.at[1, pl.ds(_off(0), CH_A)], dstv.at[0])
    _start_gather(0, 0)
    pltpu.async_copy(adj_hbm.at[1, pl.ds(_off(1), CH_A)], dstv.at[1], semd[1])
    pltpu.async_copy(adj_hbm.at[0, pl.ds(_off(0), CH_A)], srcv.at[0], semsr[0])
    pltpu.async_copy(adj_hbm.at[0, pl.ds(_off(1), CH_A)], srcv.at[1], semsr[1])

    ceil4 = ((cnt + 3) // 4) * 4

    @pl.loop(0, ceil4, step=4)
    def _ring(g):
        for b in range(4):
            i = g + b
            b2 = b % 2
            nb2 = (b + 1) % 2

            @pl.when(i + 1 < cnt)
            def _():
                pltpu.make_async_copy(
                    adj_hbm.at[1, pl.ds(0, CH_A)],
                    dstv.at[nb2], semd[nb2]).wait()
                _start_gather(i + 1, nb2)

            @pl.when(i < cnt)
            def _():
                _wait_gather(b2)

            @pl.when((i >= 2) & (i < cnt))
            def _():
                _wait_scatter(b2)

            @pl.when(i < cnt)
            def _():
                pltpu.make_async_copy(
                    adj_hbm.at[0, pl.ds(0, CH_A)],
                    srcv.at[b], semsr[b2]).wait()

            @pl.when(i + 2 < cnt)
            def _():
                pltpu.async_copy(adj_hbm.at[1, pl.ds(_off(i + 2), CH_A)],
                                 dstv.at[b2], semd[b2])
                pltpu.async_copy(adj_hbm.at[0, pl.ds(_off(i + 2), CH_A)],
                                 srcv.at[(b + 2) % 4], semsr[b2])

            @pl.when(i < cnt)
            def _():
                @pl.loop(0, CH_A)
                def _edge(row):
                    av = plsc.load_gather(
                        attv.at[b2], [jnp.full((16,), row, _i32)])
                    vs = [vfbuf[b2, row, pl.ds(j * 16, 16)]
                          for j in range(F // 16)]
                    for j in range(F // 16):
                        msgbuf[b2, row, pl.ds(j * 16, 16)] = av * vs[j]

                pltpu.async_copy(msgbuf.at[b2], acc_shared.at[srcv.at[b]],
                                 sems[b2], add=True)

    _wait_scatter(0)
    _wait_scatter(1)

    plsc.subcore_barrier()
    pltpu.sync_copy(
        acc_shared.at[pl.ds(base, 624), :],
        outp_hbm.at[cid, pl.ds(base, 624), :])

    @pl.when(sid == 15)
    def _dtail():
        pltpu.sync_copy(
            acc_shared.at[pl.ds(9984, 16), :],
            outp_hbm.at[cid, pl.ds(9984, 16), :])


def kernel(x, adj, Wq, bq, Wk, bk, Wv, bv):
    perm = jnp.asarray([(j % H) * D + j // H for j in range(F)], dtype=_i32)
    w = jnp.concatenate([Wq * SCALE, Wk, Wv[perm, :]], axis=0)
    b = jnp.concatenate([bq * SCALE, bk, bv[perm]], axis=0).reshape(1, 3 * F)

    q, k, vf = _projections(x, w.T, b)
    ex, den_parts = _edge_scores(q, k, adj)
    recip = _recip(den_parts).reshape(N * H)
    att = _normalize(ex, recip, adj)
    out_parts = _aggregate(att, vf, adj)
    out = _sum2(out_parts)
    return (out, att, att)
